# baseline (speedup 1.0000x reference)
import functools

import jax
import jax.numpy as jnp
from jax import lax
from jax.experimental import pallas as pl
from jax.experimental.pallas import tpu as pltpu
from jax.experimental.pallas import tpu_sc as plsc

B = 16384
DIM = 64
NUM_ROWS = 1000000
BAND = 128
NFULL = NUM_ROWS // BAND
EDGE0 = NFULL * BAND
NUM_CORES = 2
NUM_SUBCORES = 16
NW = NUM_CORES * NUM_SUBCORES
RPW = B // NW
LANES = 16
SUPER = 3
NSUPER = 82
SENT_U = B
SENT_IJ = 2 * B
PW = 128


def _scan_idx(idx_hbm, xbuf, hl, cnt, pos_base, lo_w, nb_w):
    lanes = lax.iota(jnp.int32, LANES)
    c127 = jnp.full((LANES,), 127, jnp.int32)
    lo_v = jnp.full((LANES,), 1, jnp.int32) * lo_w
    nb_v = jnp.full((LANES,), 1, jnp.int32) * nb_w
    pltpu.sync_copy(idx_hbm, xbuf)

    def scan_chunk(c, cnt2):
        v = xbuf[pl.ds(c * LANES, LANES)]
        pos = lanes + (c * LANES + pos_base)
        bl = lax.shift_right_logical(v, 7) - lo_v
        m = (bl >= 0) & (bl < nb_v)
        h = (lax.shift_left(bl, 22) |
             lax.shift_left(v & c127, 15) | pos)
        plsc.store_compressed(hl.at[pl.ds(cnt2, LANES)], h, mask=m)
        nhit = plsc.all_reduce_population_count(m)
        return cnt2 + nhit[0]

    return lax.fori_loop(0, B // LANES, scan_chunk, cnt)


def _extract(table_hbm, edge_hbm, g_hbm, hl, rb, arena, fsem, ssem,
             wid, lo_w, nch, sent):
    lanes = lax.iota(jnp.int32, LANES)
    c127 = jnp.full((LANES,), 127, jnp.int32)
    sent_v = jnp.full((LANES,), sent, jnp.int32)

    def process_bands(blg0, par, fcount, nb):
        parv = jnp.full((LANES,), 1, jnp.int32) * par
        b0v = jnp.full((LANES,), 1, jnp.int32) * blg0

        def chunk(c, fc):
            hc = hl[pl.ds(c * LANES, LANES)]
            blf = lax.shift_right_logical(hc, 22)
            msup = (blf >= b0v) & (blf < b0v + nb)

            def do_super(fc1):
                col = lax.shift_right_logical(hc, 15) & c127
                pos = hc & jnp.full((LANES,), 0x7FFF, jnp.int32)
                for sl in range(SUPER):
                    m = blf == b0v + sl

                    def do_band(fc2):
                        slot = lax.rem(fc2, 4)
                        arow = slot * LANES + lanes
                        colv = col + sl * BAND

                        @pl.when(fc2 >= 4)
                        def _():
                            pltpu.make_async_copy(
                                g_hbm.at[pl.ds(0, LANES), :],
                                arena.at[pl.ds(0, LANES), :], ssem).wait()

                        for f in range(DIM):
                            fv = jnp.full((LANES,), f, jnp.int32)
                            vals = plsc.load_gather(rb, [parv, fv, colv])
                            plsc.store_scatter(arena, [arow, fv], vals)
                        psel = jnp.where(m, pos, sent_v)
                        pltpu.async_copy(
                            arena.at[pl.ds(slot * LANES, LANES), :],
                            g_hbm.at[psel], ssem)
                        return fc2 + 1

                    fc1 = lax.cond(jnp.any(m), do_band, lambda x: x, fc1)
                return fc1

            return lax.cond(jnp.any(msup), do_super, lambda x: x, fc)

        return lax.fori_loop(0, nch, chunk, fcount)

    def fetch(s, par):
        start = jnp.minimum(lo_w + s * SUPER, NFULL - SUPER)
        for fg in range(8):
            pltpu.async_copy(
                table_hbm.at[pl.ds(fg * 8, 8),
                             pl.ds(start * BAND, SUPER * BAND)],
                rb.at[par, pl.ds(fg * 8, 8), :], fsem)

    fetch(jnp.int32(0), jnp.int32(0))

    def super_step(s, fcount):
        par = lax.rem(s, 2)
        pltpu.make_async_copy(
            table_hbm.at[pl.ds(0, DIM), pl.ds(0, SUPER * BAND)],
            rb.at[0], fsem).wait()

        @pl.when(s + 1 < NSUPER)
        def _():
            fetch(s + 1, lax.rem(s + 1, 2))

        start = jnp.minimum(lo_w + s * SUPER, NFULL - SUPER)
        return process_bands(start - lo_w, par, fcount, SUPER)

    fcount = lax.fori_loop(0, NSUPER, super_step, jnp.int32(0))

    @pl.when(wid == NW - 1)
    def _():
        pltpu.sync_copy(edge_hbm, rb.at[0, :, pl.ds(0, BAND)])

    fcount = lax.cond(
        wid == NW - 1,
        lambda fc: process_bands(NFULL - lo_w, jnp.int32(0), fc, 1),
        lambda fc: fc,
        fcount)

    def drain(_, carry):
        pltpu.make_async_copy(
            g_hbm.at[pl.ds(0, LANES), :],
            arena.at[pl.ds(0, LANES), :], ssem).wait()
        return carry

    lax.fori_loop(0, jnp.minimum(fcount, 4), drain, 0)


def _body_a(u_hbm, i_hbm, j_hbm, ufT, ifT, ufE, ifE,
            gu_hbm, gij_hbm, xbuf, hl, rb, arena, fsem, ssem):
    wid = lax.axis_index("s") * NUM_CORES + lax.axis_index("c")
    lo_w = wid * 244 + jnp.minimum(wid, 5)
    nb_w = jnp.where(wid < 5, 245, 244)

    padv = jnp.full((LANES,), 1023 << 22, jnp.int32)

    cnt = _scan_idx(u_hbm, xbuf, hl, jnp.int32(0), 0, lo_w, nb_w)
    hl[pl.ds(cnt, LANES)] = padv
    nch = lax.shift_right_logical(cnt + (LANES - 1), 4)
    _extract(ufT, ufE, gu_hbm, hl, rb, arena, fsem, ssem, wid, lo_w,
             nch, SENT_U)

    cnt = _scan_idx(i_hbm, xbuf, hl, jnp.int32(0), 0, lo_w, nb_w)
    cnt = _scan_idx(j_hbm, xbuf, hl, cnt, B, lo_w, nb_w)
    hl[pl.ds(cnt, LANES)] = padv
    nch = lax.shift_right_logical(cnt + (LANES - 1), 4)
    _extract(ifT, ifE, gij_hbm, hl, rb, arena, fsem, ssem, wid, lo_w,
             nch, SENT_IJ)


def _body_b(i_hbm, j_hbm, gu_hbm, gij_hbm, bias_hbm, out_hbm,
            ii, ji, bu, bv, bw, bi, bj, ov, sem):
    wid = lax.axis_index("s") * NUM_CORES + lax.axis_index("c")
    base = wid * RPW
    pltpu.sync_copy(i_hbm.at[pl.ds(base, RPW)], ii)
    pltpu.sync_copy(j_hbm.at[pl.ds(base, RPW)], ji)
    c4 = pltpu.async_copy(bias_hbm.at[ii], bi, sem)
    c5 = pltpu.async_copy(bias_hbm.at[ji], bj, sem)
    c4.wait()
    c5.wait()

    lanes = lax.iota(jnp.int32, LANES)
    HALF = RPW // 2

    for h in range(2):
        hb = base + h * HALF
        pltpu.sync_copy(gu_hbm.at[pl.ds(hb, HALF), :], bu)
        pltpu.sync_copy(gij_hbm.at[pl.ds(hb, HALF), :], bv)
        pltpu.sync_copy(gij_hbm.at[pl.ds(B + hb, HALF), :], bw)

        def group(g, carry):
            gb = g * LANES
            row_idx = lanes + gb
            ob = h * HALF + gb
            acc = bi[pl.ds(ob, LANES)] - bj[pl.ds(ob, LANES)]

            def dstep(d, a):
                dv = jnp.full((LANES,), d, jnp.int32)
                uu = plsc.load_gather(bu, [row_idx, dv])
                xi = plsc.load_gather(bv, [row_idx, dv])
                xj = plsc.load_gather(bw, [row_idx, dv])
                return a + uu * (xi - xj)

            acc = lax.fori_loop(0, DIM, dstep, acc, unroll=8)
            ov[pl.ds(ob, LANES)] = acc
            return carry

        lax.fori_loop(0, HALF // LANES, group, 0)

    pltpu.sync_copy(ov, out_hbm.at[pl.ds(base, RPW)])


@functools.partial(jax.jit, static_argnames=())
def kernel(u, i, j, user_factors, item_factors, item_biases):
    mesh = plsc.VectorSubcoreMesh(core_axis_name="c", subcore_axis_name="s")
    cp = pltpu.CompilerParams(needs_layout_passes=False)

    ka = functools.partial(
        pl.kernel,
        mesh=mesh,
        compiler_params=cp,
        out_type=(
            jax.ShapeDtypeStruct((B + LANES, PW), jnp.float32),
            jax.ShapeDtypeStruct((2 * B + LANES, PW), jnp.float32),
        ),
        scratch_types=[
            pltpu.VMEM((B,), jnp.int32),
            pltpu.VMEM((2 * B + LANES,), jnp.int32),
            pltpu.VMEM((2, DIM, SUPER * BAND), jnp.float32),
            pltpu.VMEM((4 * LANES, PW), jnp.float32),
            pltpu.SemaphoreType.DMA,
            pltpu.SemaphoreType.DMA,
        ],
    )(_body_a)

    kb = functools.partial(
        pl.kernel,
        mesh=mesh,
        compiler_params=cp,
        out_type=jax.ShapeDtypeStruct((B,), jnp.float32),
        scratch_types=[
            pltpu.VMEM((RPW,), jnp.int32),
            pltpu.VMEM((RPW,), jnp.int32),
            pltpu.VMEM((RPW // 2, PW), jnp.float32),
            pltpu.VMEM((RPW // 2, PW), jnp.float32),
            pltpu.VMEM((RPW // 2, PW), jnp.float32),
            pltpu.VMEM((RPW,), jnp.float32),
            pltpu.VMEM((RPW,), jnp.float32),
            pltpu.VMEM((RPW,), jnp.float32),
            pltpu.SemaphoreType.DMA,
        ],
    )(_body_b)

    ufT = user_factors.T
    ifT = item_factors.T
    ufE = jnp.pad(user_factors[EDGE0:].T, ((0, 0), (0, BAND - DIM)))
    ifE = jnp.pad(item_factors[EDGE0:].T, ((0, 0), (0, BAND - DIM)))
    bias_flat = item_biases.reshape(-1)
    gu, gij = ka(u, i, j, ufT, ifT, ufE, ifE)
    return kb(i, j, gu, gij, bias_flat)


# speedup vs baseline: 23.5761x; 23.5761x over previous
import functools

import jax
import jax.numpy as jnp
from jax import lax
from jax.experimental import pallas as pl
from jax.experimental.pallas import tpu as pltpu
from jax.experimental.pallas import tpu_sc as plsc

B = 16384
DIM = 64
NUM_ROWS = 1000000
NUM_CORES = 2
NUM_SUBCORES = 16
NW = NUM_CORES * NUM_SUBCORES
RPW = B // NW
LANES = 16
GROUPS = RPW // LANES


def _body(u_hbm, i_hbm, j_hbm, uf_hbm, if_hbm, bias_hbm, out_hbm,
          ui, ii, ji, uv, iv, jv, bi, bj, ov, sem):
    wid = lax.axis_index("s") * NUM_CORES + lax.axis_index("c")
    base = wid * RPW

    pltpu.sync_copy(u_hbm.at[pl.ds(base, RPW)], ui)
    pltpu.sync_copy(i_hbm.at[pl.ds(base, RPW)], ii)
    pltpu.sync_copy(j_hbm.at[pl.ds(base, RPW)], ji)

    c4 = pltpu.async_copy(bias_hbm.at[ii], bi, sem)
    c5 = pltpu.async_copy(bias_hbm.at[ji], bj, sem)

    dimv = jnp.full((LANES,), DIM, jnp.int32)

    def fire(g, carry):
        gb = g * LANES
        uo = ui[pl.ds(gb, LANES)] * dimv
        io = ii[pl.ds(gb, LANES)] * dimv
        jo = ji[pl.ds(gb, LANES)] * dimv
        for l in range(LANES):
            dst = pl.ds((gb + l) * DIM, DIM)
            uol = pl.multiple_of(uo[l], DIM)
            iol = pl.multiple_of(io[l], DIM)
            jol = pl.multiple_of(jo[l], DIM)
            pltpu.async_copy(uf_hbm.at[pl.ds(uol, DIM)], uv.at[dst], sem)
            pltpu.async_copy(if_hbm.at[pl.ds(iol, DIM)], iv.at[dst], sem)
            pltpu.async_copy(if_hbm.at[pl.ds(jol, DIM)], jv.at[dst], sem)
        return carry

    lax.fori_loop(0, GROUPS, fire, 0)

    pltpu.make_async_copy(uf_hbm.at[pl.ds(0, RPW * DIM)], uv, sem).wait()
    pltpu.make_async_copy(uf_hbm.at[pl.ds(0, RPW * DIM)], iv, sem).wait()
    pltpu.make_async_copy(uf_hbm.at[pl.ds(0, RPW * DIM)], jv, sem).wait()
    c4.wait()
    c5.wait()

    lanes = lax.iota(jnp.int32, LANES)

    def group(g, carry):
        rb = g * LANES
        flat_base = lax.mul(lanes + rb, jnp.full((LANES,), DIM, jnp.int32))
        acc = bi[pl.ds(rb, LANES)] - bj[pl.ds(rb, LANES)]

        def dstep(d, a):
            fidx = flat_base + jnp.full((LANES,), d, jnp.int32)
            uu = plsc.load_gather(uv, [fidx])
            xi = plsc.load_gather(iv, [fidx])
            xj = plsc.load_gather(jv, [fidx])
            return a + uu * (xi - xj)

        acc = lax.fori_loop(0, DIM, dstep, acc, unroll=8)
        ov[pl.ds(rb, LANES)] = acc
        return carry

    lax.fori_loop(0, GROUPS, group, 0)
    pltpu.sync_copy(ov, out_hbm.at[pl.ds(base, RPW)])


@functools.partial(jax.jit, static_argnames=())
def kernel(u, i, j, user_factors, item_factors, item_biases):
    mesh = plsc.VectorSubcoreMesh(core_axis_name="c", subcore_axis_name="s")
    k = functools.partial(
        pl.kernel,
        mesh=mesh,
        compiler_params=pltpu.CompilerParams(needs_layout_passes=False),
        out_type=jax.ShapeDtypeStruct((B,), jnp.float32),
        scratch_types=[
            pltpu.VMEM((RPW,), jnp.int32),
            pltpu.VMEM((RPW,), jnp.int32),
            pltpu.VMEM((RPW,), jnp.int32),
            pltpu.VMEM((RPW * DIM,), jnp.float32),
            pltpu.VMEM((RPW * DIM,), jnp.float32),
            pltpu.VMEM((RPW * DIM,), jnp.float32),
            pltpu.VMEM((RPW,), jnp.float32),
            pltpu.VMEM((RPW,), jnp.float32),
            pltpu.VMEM((RPW,), jnp.float32),
            pltpu.SemaphoreType.DMA,
        ],
    )(_body)
    uf1 = user_factors.reshape(-1)
    if1 = item_factors.reshape(-1)
    bias_flat = item_biases.reshape(-1)
    return k(u, i, j, uf1, if1, bias_flat)
